# Initial kernel scaffold; baseline (speedup 1.0000x reference)
import jax
import jax.numpy as jnp
from jax.experimental import pallas as pl


def kernel(x, pos_embedding):
    raise NotImplementedError("write your pallas kernel here")


# speedup vs baseline: 1.6797x; 1.6797x over previous
import jax
import jax.numpy as jnp
from jax.experimental import pallas as pl

_SEQ_BLOCK = 1024


def _add_pos_kernel(x_ref, pos_ref, out_ref):
    out_ref[0] = x_ref[0] + pos_ref[...]


def kernel(x, pos_embedding):
    batch, seq, d = x.shape
    pos = pos_embedding[:seq]
    blk = min(_SEQ_BLOCK, seq)
    grid = (seq // blk, batch)
    return pl.pallas_call(
        _add_pos_kernel,
        grid=grid,
        in_specs=[
            pl.BlockSpec((1, blk, d), lambda i, j: (j, i, 0)),
            pl.BlockSpec((blk, d), lambda i, j: (i, 0)),
        ],
        out_specs=pl.BlockSpec((1, blk, d), lambda i, j: (j, i, 0)),
        out_shape=jax.ShapeDtypeStruct((batch, seq, d), x.dtype),
    )(x, pos)


# speedup vs baseline: 1.8010x; 1.0722x over previous
import jax
import jax.numpy as jnp
from jax.experimental import pallas as pl

_SEQ_BLOCK = 2048


def _add_pos_kernel(x_ref, pos_ref, out_ref):
    out_ref[0] = x_ref[0] + pos_ref[...]


def kernel(x, pos_embedding):
    batch, seq, d = x.shape
    pos = pos_embedding[:seq]
    blk = min(_SEQ_BLOCK, seq)
    grid = (seq // blk, batch)
    return pl.pallas_call(
        _add_pos_kernel,
        grid=grid,
        in_specs=[
            pl.BlockSpec((1, blk, d), lambda i, j: (j, i, 0)),
            pl.BlockSpec((blk, d), lambda i, j: (i, 0)),
        ],
        out_specs=pl.BlockSpec((1, blk, d), lambda i, j: (j, i, 0)),
        out_shape=jax.ShapeDtypeStruct((batch, seq, d), x.dtype),
    )(x, pos)
